# baseline (speedup 1.0000x reference)
import functools

import jax
import jax.numpy as jnp
from jax import lax
from jax.experimental import pallas as pl
from jax.experimental.pallas import tpu as pltpu
from jax.experimental.pallas import tpu_sc as plsc

_NROWS = 26
_SUBS_PER_ROW = 32
_W = 16384
_TOTAL = _NROWS * _SUBS_PER_ROW
_NW = 32
_PER_W = _TOTAL // _NW
_NBUF = 7
_GD = 4

_mesh = plsc.VectorSubcoreMesh(core_axis_name="c", subcore_axis_name="s")


@functools.partial(
    pl.kernel,
    out_type=jax.ShapeDtypeStruct((_TOTAL, _W), jnp.float32),
    mesh=_mesh,
    scratch_types=[
        pltpu.VMEM((32,), jnp.int32),
        pltpu.VMEM((_NBUF * _W,), jnp.float32),
        pltpu.SemaphoreType.DMA((_NBUF,)),
        pltpu.SemaphoreType.DMA((_NBUF,)),
    ],
)
def _sc_gather(table, idx_hbm, out, idxv, buf, gsem, psem):
    wid = lax.axis_index("s") * 2 + lax.axis_index("c")
    base = wid * _PER_W
    pltpu.sync_copy(idx_hbm.at[wid], idxv)
    lo = idxv[pl.ds(0, 16)]
    hi = idxv[pl.ds(16, 16)]

    def slot(k):
        return buf.at[pl.ds((k % _NBUF) * _W, _W)]

    def gather(k):
        s = lo[k] if k < 16 else hi[k - 16]
        return pltpu.make_async_copy(table.at[s], slot(k), gsem.at[k % _NBUF])

    def put(k):
        return pltpu.make_async_copy(slot(k), out.at[base + k], psem.at[k % _NBUF])

    started_g = min(_GD, _PER_W)
    waited_p = 0
    for k in range(started_g):
        gather(k).start()
    for k in range(_PER_W):
        gather(k).wait()
        put(k).start()
        if started_g < _PER_W:
            need_free = started_g - _NBUF
            while waited_p <= need_free:
                put(waited_p).wait()
                waited_p += 1
            gather(started_g).start()
            started_g += 1
    while waited_p < _PER_W:
        put(waited_p).wait()
        waited_p += 1


def kernel(mamdani_output, mapping):
    src = jnp.transpose(mamdani_output, (0, 2, 1)).reshape(3200, _W)
    t = jnp.arange(_NW * 32, dtype=jnp.int32).reshape(_NW, 32)
    t = jnp.minimum((t // 32) * _PER_W + (t % 32), _TOTAL - 1)
    idx = mapping.reshape(_NROWS)[t >> 5] * _SUBS_PER_ROW + (t & 31)
    out = _sc_gather(src, idx)
    out = jnp.transpose(out.reshape(_NROWS, _SUBS_PER_ROW, _W), (0, 2, 1))
    return jnp.expand_dims(out, 1)


# speedup vs baseline: 1.0456x; 1.0456x over previous
import functools

import jax
import jax.numpy as jnp
from jax import lax
from jax.experimental import pallas as pl
from jax.experimental.pallas import tpu as pltpu
from jax.experimental.pallas import tpu_sc as plsc

_NROWS = 26
_W = 16384
_TOTAL_TR = _NROWS * 4
_NSLOT = 15
_NACT = 2 * _NSLOT
_KMAX = 4

_mesh = plsc.VectorSubcoreMesh(core_axis_name="c", subcore_axis_name="s")


@functools.partial(
    pl.kernel,
    out_type=jax.ShapeDtypeStruct((_TOTAL_TR * 8, _W), jnp.float32),
    mesh=_mesh,
    scratch_types=[
        pltpu.VMEM((16,), jnp.int32),
        pltpu.VMEM_SHARED((_NSLOT, 8, _W), jnp.float32),
        pltpu.SemaphoreType.DMA,
        pltpu.SemaphoreType.DMA,
    ],
)
def _sc_gather(table, idx_hbm, out, idxv, shared, gsem, psem):
    sid = lax.axis_index("s")
    cid = lax.axis_index("c")
    vid = cid * _NSLOT + sid

    @pl.when(sid < _NSLOT)
    def _():
        pltpu.sync_copy(idx_hbm.at[vid], idxv)
        srows = idxv[...]
        slot = shared.at[sid]
        for k in range(_KMAX):
            u = vid + _NACT * k

            @pl.when(u < _TOTAL_TR)
            def _():
                s = srows[k]
                pltpu.make_async_copy(
                    table.at[pl.ds(s * 8, 8), :], slot, gsem
                ).start()
                pltpu.make_async_copy(
                    table.at[pl.ds(s * 8, 8), :], slot, gsem
                ).wait()
                pltpu.make_async_copy(
                    slot, out.at[pl.ds(u * 8, 8), :], psem
                ).start()
                pltpu.make_async_copy(
                    slot, out.at[pl.ds(u * 8, 8), :], psem
                ).wait()


def kernel(mamdani_output, mapping):
    src = jnp.transpose(mamdani_output, (0, 2, 1)).reshape(3200, _W)
    v = jnp.arange(32, dtype=jnp.int32)[:, None]
    k = jnp.arange(16, dtype=jnp.int32)[None, :]
    u = jnp.minimum(v + _NACT * k, _TOTAL_TR - 1)
    idx = mapping.reshape(_NROWS)[u // 4] * 4 + (u % 4)
    out = _sc_gather(src, idx)
    out = jnp.transpose(out.reshape(_NROWS, 32, _W), (0, 2, 1))
    return jnp.expand_dims(out, 1)
